# Initial kernel scaffold; baseline (speedup 1.0000x reference)
import jax
import jax.numpy as jnp
from jax.experimental import pallas as pl


def kernel(input_ids, emb, W1, b1, W2, b2):
    raise NotImplementedError("write your pallas kernel here")


# speedup vs baseline: 4.9972x; 4.9972x over previous
import functools

import jax
import jax.numpy as jnp
from jax import lax
from jax.experimental import pallas as pl
from jax.experimental.pallas import tpu as pltpu
from jax.experimental.pallas import tpu_sc as plsc

GATHER_WINDOW = 128


def _sc_gather(emb, flat_ids):
    n = flat_ids.shape[0]
    d = emb.shape[1]
    mesh = plsc.VectorSubcoreMesh(core_axis_name="c", subcore_axis_name="s")
    ids2 = flat_ids.reshape(1, n)

    @functools.partial(
        pl.kernel,
        out_type=jax.ShapeDtypeStruct((n, d), emb.dtype),
        mesh=mesh,
        compiler_params=pltpu.CompilerParams(use_tc_tiling_on_sc=False),
    )
    def gk(emb_hbm, ids_hbm, out_hbm):
        def body(i_vmem, o_vmem):
            pltpu.sync_copy(emb_hbm.at[i_vmem.at[0]], o_vmem)

        pltpu.emit_pipeline(
            body,
            grid=(n // GATHER_WINDOW,),
            in_specs=[pl.BlockSpec((1, GATHER_WINDOW), lambda i: (0, i))],
            out_specs=[pl.BlockSpec((GATHER_WINDOW, d), lambda i: (i, 0))],
            core_axis_name=("c", "s"),
            dimension_semantics=(pltpu.PARALLEL,),
        )(ids_hbm, out_hbm)

    return gk(emb, ids2)


def _mlp(x, W1, b1, W2, b2, block_b=512):
    bsz, k = x.shape
    hid = W1.shape[1]
    out = W2.shape[1]

    def body(x_ref, w1_ref, b1_ref, w2_ref, b2_ref, o_ref):
        xb = x_ref[...].astype(jnp.bfloat16)
        w1 = w1_ref[...].astype(jnp.bfloat16)
        h = jnp.dot(xb, w1, preferred_element_type=jnp.float32) + b1_ref[...]
        h = jnp.maximum(h, 0.0).astype(jnp.bfloat16)
        w2 = w2_ref[...].astype(jnp.bfloat16)
        o = jnp.dot(h, w2, preferred_element_type=jnp.float32) + b2_ref[...]
        o_ref[...] = jnp.maximum(o, 0.0)

    return pl.pallas_call(
        body,
        grid=(bsz // block_b,),
        in_specs=[
            pl.BlockSpec((block_b, k), lambda i: (i, 0)),
            pl.BlockSpec((k, hid), lambda i: (0, 0)),
            pl.BlockSpec((1, hid), lambda i: (0, 0)),
            pl.BlockSpec((hid, out), lambda i: (0, 0)),
            pl.BlockSpec((1, out), lambda i: (0, 0)),
        ],
        out_specs=pl.BlockSpec((block_b, out), lambda i: (i, 0)),
        out_shape=jax.ShapeDtypeStruct((bsz, out), jnp.float32),
    )(x, W1, b1, W2, b2)


def kernel(input_ids, emb, W1, b1, W2, b2):
    bsz, seq = input_ids.shape
    d = emb.shape[1]
    flat = input_ids.reshape(-1).astype(jnp.int32)
    gathered = _sc_gather(emb, flat)
    x = gathered.reshape(bsz, seq * d)
    return _mlp(x, W1, b1.reshape(1, -1), W2, b2.reshape(1, -1))


# speedup vs baseline: 5.4861x; 1.0978x over previous
import functools

import jax
import jax.numpy as jnp
from jax import lax
from jax.experimental import pallas as pl
from jax.experimental.pallas import tpu as pltpu
from jax.experimental.pallas import tpu_sc as plsc

GATHER_WINDOW = 512


def _sc_gather(emb, flat_ids):
    n = flat_ids.shape[0]
    d = emb.shape[1]
    mesh = plsc.VectorSubcoreMesh(core_axis_name="c", subcore_axis_name="s")
    ids2 = flat_ids.reshape(1, n)

    @functools.partial(
        pl.kernel,
        out_type=jax.ShapeDtypeStruct((n, d), emb.dtype),
        mesh=mesh,
        compiler_params=pltpu.CompilerParams(use_tc_tiling_on_sc=False),
    )
    def gk(emb_hbm, ids_hbm, out_hbm):
        def body(i_vmem, o_vmem):
            pltpu.sync_copy(emb_hbm.at[i_vmem.at[0]], o_vmem)

        pltpu.emit_pipeline(
            body,
            grid=(n // GATHER_WINDOW,),
            in_specs=[pl.BlockSpec((1, GATHER_WINDOW), lambda i: (0, i))],
            out_specs=[pl.BlockSpec((GATHER_WINDOW, d), lambda i: (i, 0))],
            core_axis_name=("c", "s"),
            dimension_semantics=(pltpu.PARALLEL,),
        )(ids_hbm, out_hbm)

    return gk(emb, ids2)


def _mlp(x, W1, b1, W2, b2, block_b=512):
    bsz, k = x.shape
    hid = W1.shape[1]
    out = W2.shape[1]

    def body(x_ref, w1_ref, b1_ref, w2_ref, b2_ref, o_ref):
        xb = x_ref[...].astype(jnp.bfloat16)
        w1 = w1_ref[...].astype(jnp.bfloat16)
        h = jnp.dot(xb, w1, preferred_element_type=jnp.float32) + b1_ref[...]
        h = jnp.maximum(h, 0.0).astype(jnp.bfloat16)
        w2 = w2_ref[...].astype(jnp.bfloat16)
        o = jnp.dot(h, w2, preferred_element_type=jnp.float32) + b2_ref[...]
        o_ref[...] = jnp.maximum(o, 0.0)

    return pl.pallas_call(
        body,
        grid=(bsz // block_b,),
        in_specs=[
            pl.BlockSpec((block_b, k), lambda i: (i, 0)),
            pl.BlockSpec((k, hid), lambda i: (0, 0)),
            pl.BlockSpec((1, hid), lambda i: (0, 0)),
            pl.BlockSpec((hid, out), lambda i: (0, 0)),
            pl.BlockSpec((1, out), lambda i: (0, 0)),
        ],
        out_specs=pl.BlockSpec((block_b, out), lambda i: (i, 0)),
        out_shape=jax.ShapeDtypeStruct((bsz, out), jnp.float32),
        compiler_params=pltpu.CompilerParams(dimension_semantics=("parallel",)),
    )(x, W1, b1, W2, b2)


def kernel(input_ids, emb, W1, b1, W2, b2):
    bsz, seq = input_ids.shape
    d = emb.shape[1]
    flat = input_ids.reshape(-1).astype(jnp.int32)
    gathered = _sc_gather(emb, flat)
    x = gathered.reshape(bsz, seq * d)
    return _mlp(x, W1, b1.reshape(1, -1), W2, b2.reshape(1, -1))
